# Initial kernel scaffold; baseline (speedup 1.0000x reference)
import jax
import jax.numpy as jnp
from jax.experimental import pallas as pl


def kernel(freqs_cis, seqlen, tok_idx):
    raise NotImplementedError("write your pallas kernel here")


# speedup vs baseline: 5.5364x; 5.5364x over previous
import functools

import jax
import jax.numpy as jnp
from jax import lax
from jax.experimental import pallas as pl
from jax.experimental.pallas import tpu as pltpu
from jax.experimental.pallas import tpu_sc as plsc

_ROW = 256
_CHUNK = 128


@functools.lru_cache(maxsize=None)
def _make_gather(n_rows, n_vocab):
    info = plsc.get_sparse_core_info()
    nw = info.num_cores * info.num_subcores
    assert n_rows % (nw * _CHUNK) == 0
    rows_per_w = n_rows // nw
    n_chunks = rows_per_w // _CHUNK

    mesh = plsc.VectorSubcoreMesh(core_axis_name="c", subcore_axis_name="s")

    @functools.partial(
        pl.kernel,
        mesh=mesh,
        out_type=jax.ShapeDtypeStruct((n_rows, _ROW), jnp.float32),
        scratch_types=[
            pltpu.VMEM((n_chunks, _CHUNK), jnp.int32),
            pltpu.VMEM((_CHUNK, _ROW), jnp.float32),
            pltpu.SemaphoreType.DMA,
        ],
    )
    def gather(table_hbm, idx_hbm, out_hbm, idx_v, rows_v, sem):
        wid = lax.axis_index("s") * info.num_cores + lax.axis_index("c")
        base = wid * rows_per_w
        pltpu.sync_copy(idx_hbm.at[wid], idx_v)

        def chunk(i, carry):
            pltpu.async_copy(table_hbm.at[idx_v.at[i]], rows_v, sem).wait()
            pltpu.sync_copy(rows_v, out_hbm.at[pl.ds(base + i * _CHUNK, _CHUNK)])
            return carry

        lax.fori_loop(0, n_chunks, chunk, 0)

    return gather


def kernel(freqs_cis, seqlen, tok_idx):
    del seqlen
    b, s = tok_idx.shape
    v = freqs_cis.shape[0]
    table = freqs_cis.reshape(v, _ROW)
    info = plsc.get_sparse_core_info()
    nw = info.num_cores * info.num_subcores
    n_rows = b * s
    idx = tok_idx.astype(jnp.int32).reshape(nw, n_rows // (nw * _CHUNK), _CHUNK)
    out = _make_gather(n_rows, v)(table, idx)
    return out.reshape(b, s, freqs_cis.shape[1], 2, 2)


# speedup vs baseline: 5.7178x; 1.0328x over previous
import functools

import jax
import jax.numpy as jnp
from jax import lax
from jax.experimental import pallas as pl
from jax.experimental.pallas import tpu as pltpu
from jax.experimental.pallas import tpu_sc as plsc

_ROW = 256
_CHUNK = 128


@functools.lru_cache(maxsize=None)
def _make_gather(n_rows, n_vocab):
    info = plsc.get_sparse_core_info()
    nw = info.num_cores * info.num_subcores
    assert n_rows % (nw * _CHUNK) == 0
    rows_per_w = n_rows // nw
    n_chunks = rows_per_w // _CHUNK

    mesh = plsc.VectorSubcoreMesh(core_axis_name="c", subcore_axis_name="s")

    assert n_chunks % 2 == 0 and n_chunks >= 4

    @functools.partial(
        pl.kernel,
        mesh=mesh,
        out_type=jax.ShapeDtypeStruct((n_rows, _ROW), jnp.float32),
        scratch_types=[
            pltpu.VMEM((n_chunks, _CHUNK), jnp.int32),
            pltpu.VMEM((_CHUNK, _ROW), jnp.float32),
            pltpu.VMEM((_CHUNK, _ROW), jnp.float32),
            pltpu.SemaphoreType.DMA,
            pltpu.SemaphoreType.DMA,
            pltpu.SemaphoreType.DMA,
            pltpu.SemaphoreType.DMA,
        ],
    )
    def gather(table_hbm, idx_hbm, out_hbm, idx_v, rows0, rows1, g0, g1, w0, w1):
        wid = lax.axis_index("s") * info.num_cores + lax.axis_index("c")
        base = wid * rows_per_w
        pltpu.sync_copy(idx_hbm.at[wid], idx_v)

        def sg(i, buf, sem):
            pltpu.async_copy(table_hbm.at[idx_v.at[i]], buf, sem)

        def sw(i, buf, sem):
            pltpu.async_copy(buf, out_hbm.at[pl.ds(base + i * _CHUNK, _CHUNK)], sem)

        def wait_g(buf, sem):
            pltpu.make_async_copy(table_hbm.at[pl.ds(0, _CHUNK)], buf, sem).wait()

        def wait_w(buf, sem):
            pltpu.make_async_copy(buf, out_hbm.at[pl.ds(base, _CHUNK)], sem).wait()

        sg(0, rows0, g0)
        sg(1, rows1, g1)
        wait_g(rows0, g0)
        sw(0, rows0, w0)
        wait_g(rows1, g1)
        sw(1, rows1, w1)

        def group(g, carry):
            i0, i1 = 2 * g, 2 * g + 1
            wait_w(rows0, w0)
            sg(i0, rows0, g0)
            wait_w(rows1, w1)
            sg(i1, rows1, g1)
            wait_g(rows0, g0)
            sw(i0, rows0, w0)
            wait_g(rows1, g1)
            sw(i1, rows1, w1)
            return carry

        lax.fori_loop(1, n_chunks // 2, group, 0)
        wait_w(rows0, w0)
        wait_w(rows1, w1)

    return gather


def kernel(freqs_cis, seqlen, tok_idx):
    del seqlen
    b, s = tok_idx.shape
    v = freqs_cis.shape[0]
    table = freqs_cis.reshape(v, _ROW)
    info = plsc.get_sparse_core_info()
    nw = info.num_cores * info.num_subcores
    n_rows = b * s
    idx = tok_idx.astype(jnp.int32).reshape(nw, n_rows // (nw * _CHUNK), _CHUNK)
    out = _make_gather(n_rows, v)(table, idx)
    return out.reshape(b, s, freqs_cis.shape[1], 2, 2)
